# Initial kernel scaffold; baseline (speedup 1.0000x reference)
import functools

import jax
import jax.numpy as jnp
from jax import lax
from jax.experimental import pallas as pl
from jax.experimental.pallas import tpu as pltpu
from jax.experimental.pallas import tpu_sc as plsc

N_NODES = 10000
N_EDGES = 10000
N_INC = 320000
D = 128
NPAD = 10240
NC = 2
NS = 16
NW = NC * NS
CHUNK = N_INC // NW
BI = 80
NB = CHUNK // BI
STRIPE = NPAD // NS
CW = 16

_mesh = plsc.VectorSubcoreMesh(core_axis_name="c", subcore_axis_name="s")


def _zero_fill_2d(buf, rows, width):
    z16 = jnp.zeros((16,), jnp.float32)

    def body(i, _):
        for k in range(width // 16):
            buf[i, pl.ds(k * 16, 16)] = z16
        return 0

    lax.fori_loop(0, rows, body, 0)


def _fill_ones_2d(buf, rows, width):
    o16 = jnp.ones((16,), jnp.float32)

    def body(i, _):
        for k in range(width // 16):
            buf[i, pl.ds(k * 16, 16)] = o16
        return 0

    lax.fori_loop(0, rows, body, 0)


def _sc_aggregate(rows_hbm, widx_hbm, ridx_hbm, acc_out, cnt_out,
                  widx2d, ridx1d, rowbuf, ones_v, zcnt, acc_sp, cnt_sp):
    c = lax.axis_index("c")
    s = lax.axis_index("s")
    w = c * NS + s
    base = w * CHUNK

    _zero_fill_2d(rowbuf, BI, D)
    _zero_fill_2d(zcnt, BI, CW)
    for r in range(STRIPE // BI):
        pltpu.sync_copy(rowbuf, acc_sp.at[pl.ds(s * STRIPE + r * BI, BI)])
        pltpu.sync_copy(zcnt, cnt_sp.at[pl.ds(s * STRIPE + r * BI, BI)])
    _fill_ones_2d(ones_v, BI, CW)

    pltpu.sync_copy(ridx_hbm.at[pl.ds(base, CHUNK)], ridx1d)
    plsc.subcore_barrier()

    def body(j, _):
        pltpu.sync_copy(widx_hbm.at[pl.ds(base + j * BI, BI)], widx2d.at[j])
        pltpu.sync_copy(rows_hbm.at[ridx1d.at[pl.ds(j * BI, BI)]], rowbuf)
        pltpu.sync_copy(rowbuf, acc_sp.at[widx2d.at[j]], add=True)
        pltpu.sync_copy(ones_v, cnt_sp.at[widx2d.at[j]], add=True)
        return 0

    lax.fori_loop(0, NB, body, 0)
    plsc.subcore_barrier()

    pltpu.sync_copy(acc_sp.at[pl.ds(s * STRIPE, STRIPE)],
                    acc_out.at[c, pl.ds(s * STRIPE, STRIPE)])
    pltpu.sync_copy(cnt_sp.at[pl.ds(s * STRIPE, STRIPE)],
                    cnt_out.at[c, pl.ds(s * STRIPE, STRIPE)])


_sc_pass = pl.kernel(
    _sc_aggregate,
    out_type=(
        jax.ShapeDtypeStruct((NC, NPAD, D), jnp.float32),
        jax.ShapeDtypeStruct((NC, NPAD, CW), jnp.float32),
    ),
    mesh=_mesh,
    scratch_types=(
        pltpu.VMEM((NB, BI), jnp.int32),
        pltpu.VMEM((CHUNK,), jnp.int32),
        pltpu.VMEM((BI, D), jnp.float32),
        pltpu.VMEM((BI, CW), jnp.float32),
        pltpu.VMEM((BI, CW), jnp.float32),
        pltpu.VMEM_SHARED((NPAD, D), jnp.float32),
        pltpu.VMEM_SHARED((NPAD, CW), jnp.float32),
    ),
)


def _tc_combine_scale(s0_ref, s1_ref, c0_ref, c1_ref, w_ref, out_ref):
    cnt = c0_ref[:] + c1_ref[:]
    inv = jnp.where(cnt > 0, 1.0 / jnp.maximum(cnt, 1.0), 0.0)
    ssum = s0_ref[:] + s1_ref[:]
    m = lax.dot_general(ssum, w_ref[:], (((1,), (1,)), ((), ())),
                        preferred_element_type=jnp.float32)
    out_ref[:] = m * inv


def _tc_finalize(o0_ref, o1_ref, c0_ref, c1_ref, bias_ref, out_ref):
    cnt = c0_ref[:] + c1_ref[:]
    inv = jnp.where(cnt > 0, 1.0 / jnp.maximum(cnt, 1.0), 0.0)
    t = (o0_ref[:] + o1_ref[:]) * inv + bias_ref[:]
    t = jnp.maximum(t, 0.0)
    rows = lax.broadcasted_iota(jnp.int32, (NPAD, D), 0)
    t = jnp.where(rows < N_NODES, t, 0.0)
    out_ref[:] = jnp.sum(t, axis=0, keepdims=True) * (1.0 / N_NODES)


def kernel(x_node_features, hyperedge_index, W, b):
    node_idx = hyperedge_index[0].astype(jnp.int32)
    edge_idx = hyperedge_index[1].astype(jnp.int32)

    s_part, b_part = _sc_pass(x_node_features, edge_idx, node_idx)

    m2 = pl.pallas_call(
        _tc_combine_scale,
        out_shape=jax.ShapeDtypeStruct((NPAD, D), jnp.float32),
    )(s_part[0], s_part[1], b_part[0, :, 0:1], b_part[1, :, 0:1], W)

    o_part, d_part = _sc_pass(m2, node_idx, edge_idx)

    out = pl.pallas_call(
        _tc_finalize,
        out_shape=jax.ShapeDtypeStruct((1, D), jnp.float32),
    )(o_part[0], o_part[1], d_part[0, :, 0:1], d_part[1, :, 0:1],
      b.reshape(1, D))
    return out.reshape(D)


# speedup vs baseline: 2.3037x; 2.3037x over previous
import jax
import jax.numpy as jnp
from jax import lax
from jax.experimental import pallas as pl
from jax.experimental.pallas import tpu as pltpu
from jax.experimental.pallas import tpu_sc as plsc

N_NODES = 10000
N_INC = 320000
D = 128
NC = 2
NS = 16
NW = NC * NS
BINS = 5120
GARB = 128
ROWS = BINS + GARB
NPAD = 2 * BINS
BI = 80
CHUNK = N_INC // NS
NB = CHUNK // BI
STRIPE = ROWS // NS
HCHUNK = N_INC // NW
HNB = HCHUNK // BI
HSTRIPE = NPAD // NS
CW = 16

_mesh = plsc.VectorSubcoreMesh(core_axis_name="c", subcore_axis_name="s")


def _zero_fill_2d(buf, rows, width):
    z16 = jnp.zeros((16,), jnp.float32)

    def body(i, _):
        for k in range(width // 16):
            buf[i, pl.ds(k * 16, 16)] = z16
        return 0

    lax.fori_loop(0, rows, body, 0)


def _sc_aggregate(rows_hbm, widx_hbm, ridx_hbm, acc_out,
                  widx2d, ridx2d, rowbuf, acc_sp):
    c = lax.axis_index("c")
    s = lax.axis_index("s")
    lo = c * BINS

    _zero_fill_2d(rowbuf, BI, D)

    def zstripe(r, _):
        pltpu.sync_copy(rowbuf, acc_sp.at[pl.ds(s * STRIPE + r * BI, BI)])
        return 0

    lax.fori_loop(0, STRIPE // BI, zstripe, 0)
    tail = STRIPE - (STRIPE // BI) * BI
    toff = s * STRIPE + (STRIPE // BI) * BI
    if tail:
        pltpu.sync_copy(rowbuf.at[pl.ds(0, tail)],
                        acc_sp.at[pl.ds(toff, tail)])

    pltpu.sync_copy(widx_hbm.at[s], widx2d)
    pltpu.sync_copy(ridx_hbm.at[s], ridx2d)

    def remap(j, _):
        for k in range(BI // 16):
            idx = widx2d[j, pl.ds(k * 16, 16)]
            local = idx - lo
            oob = (local < 0) | (local >= BINS)
            garb = BINS + (idx & (GARB - 1))
            widx2d[j, pl.ds(k * 16, 16)] = jnp.where(oob, garb, local)
        return 0

    lax.fori_loop(0, NB, remap, 0)
    plsc.subcore_barrier()

    def step(j, _):
        pltpu.sync_copy(rows_hbm.at[ridx2d.at[j]], rowbuf)
        pltpu.sync_copy(rowbuf, acc_sp.at[widx2d.at[j]], add=True)
        return 0

    lax.fori_loop(0, NB, step, 0)
    plsc.subcore_barrier()

    def bounce(r, _):
        pltpu.sync_copy(acc_sp.at[pl.ds(s * STRIPE + r * BI, BI)], rowbuf)
        pltpu.sync_copy(rowbuf, acc_out.at[c, pl.ds(s * STRIPE + r * BI, BI)])
        return 0

    lax.fori_loop(0, STRIPE // BI, bounce, 0)
    if tail:
        pltpu.sync_copy(acc_sp.at[pl.ds(toff, tail)],
                        rowbuf.at[pl.ds(0, tail)])
        pltpu.sync_copy(rowbuf.at[pl.ds(0, tail)],
                        acc_out.at[c, pl.ds(toff, tail)])


_sc_pass = pl.kernel(
    _sc_aggregate,
    out_type=jax.ShapeDtypeStruct((NC, ROWS, D), jnp.float32),
    mesh=_mesh,
    scratch_types=(
        pltpu.VMEM((NB, BI), jnp.int32),
        pltpu.VMEM((NB, BI), jnp.int32),
        pltpu.VMEM((BI, D), jnp.float32),
        pltpu.VMEM_SHARED((ROWS, D), jnp.float32),
    ),
)


def _sc_hist(eidx_hbm, nidx_hbm, ecnt_out, ncnt_out,
             eidx2d, nidx2d, ones_v, ecnt_sp, ncnt_sp):
    c = lax.axis_index("c")
    s = lax.axis_index("s")
    w = c * NS + s

    _zero_fill_2d(ones_v, BI, CW)

    def zstripe(r, _):
        pltpu.sync_copy(ones_v, ecnt_sp.at[pl.ds(s * HSTRIPE + r * BI, BI)])
        pltpu.sync_copy(ones_v, ncnt_sp.at[pl.ds(s * HSTRIPE + r * BI, BI)])
        return 0

    lax.fori_loop(0, HSTRIPE // BI, zstripe, 0)

    o16 = jnp.ones((16,), jnp.float32)

    def fill(i, _):
        ones_v[i, pl.ds(0, 16)] = o16
        return 0

    lax.fori_loop(0, BI, fill, 0)

    pltpu.sync_copy(eidx_hbm.at[w], eidx2d)
    pltpu.sync_copy(nidx_hbm.at[w], nidx2d)
    plsc.subcore_barrier()

    def step(j, _):
        pltpu.sync_copy(ones_v, ecnt_sp.at[eidx2d.at[j]], add=True)
        pltpu.sync_copy(ones_v, ncnt_sp.at[nidx2d.at[j]], add=True)
        return 0

    lax.fori_loop(0, HNB, step, 0)
    plsc.subcore_barrier()

    def bounce(r, _):
        off = s * HSTRIPE + r * BI
        pltpu.sync_copy(ecnt_sp.at[pl.ds(off, BI)], ones_v)
        pltpu.sync_copy(ones_v, ecnt_out.at[c, pl.ds(off, BI)])
        pltpu.sync_copy(ncnt_sp.at[pl.ds(off, BI)], ones_v)
        pltpu.sync_copy(ones_v, ncnt_out.at[c, pl.ds(off, BI)])
        return 0

    lax.fori_loop(0, HSTRIPE // BI, bounce, 0)


_sc_hist_pass = pl.kernel(
    _sc_hist,
    out_type=(
        jax.ShapeDtypeStruct((NC, NPAD, CW), jnp.float32),
        jax.ShapeDtypeStruct((NC, NPAD, CW), jnp.float32),
    ),
    mesh=_mesh,
    scratch_types=(
        pltpu.VMEM((HNB, BI), jnp.int32),
        pltpu.VMEM((HNB, BI), jnp.int32),
        pltpu.VMEM((BI, CW), jnp.float32),
        pltpu.VMEM_SHARED((NPAD, CW), jnp.float32),
        pltpu.VMEM_SHARED((NPAD, CW), jnp.float32),
    ),
)


def _tc_combine_scale(s0, s1, c0, c1, w_ref, out_ref):
    ssum = jnp.concatenate([s0[:], s1[:]], axis=0)
    cnt = jnp.concatenate([c0[:], c1[:]], axis=0)
    inv = jnp.where(cnt > 0, 1.0 / jnp.maximum(cnt, 1.0), 0.0)
    m = lax.dot_general(ssum, w_ref[:], (((1,), (1,)), ((), ())),
                        preferred_element_type=jnp.float32)
    out_ref[:] = m * inv


def _tc_finalize(o0, o1, c0, c1, bias_ref, out_ref):
    t = jnp.concatenate([o0[:], o1[:]], axis=0)
    cnt = jnp.concatenate([c0[:], c1[:]], axis=0)
    inv = jnp.where(cnt > 0, 1.0 / jnp.maximum(cnt, 1.0), 0.0)
    t = t * inv + bias_ref[:]
    t = jnp.maximum(t, 0.0)
    rows = lax.broadcasted_iota(jnp.int32, (NPAD, D), 0)
    t = jnp.where(rows < N_NODES, t, 0.0)
    out_ref[:] = jnp.sum(t, axis=0, keepdims=True) * (1.0 / N_NODES)


def kernel(x_node_features, hyperedge_index, W, b):
    node_idx = hyperedge_index[0].astype(jnp.int32)
    edge_idx = hyperedge_index[1].astype(jnp.int32)
    nidx16 = node_idx.reshape(NS, NB, BI)
    eidx16 = edge_idx.reshape(NS, NB, BI)
    nidx32 = node_idx.reshape(NW, HNB, BI)
    eidx32 = edge_idx.reshape(NW, HNB, BI)

    ones_tab = jnp.ones((N_NODES, D), jnp.float32)
    ecnt = _sc_pass(ones_tab, eidx16, nidx16)
    ncnt = _sc_pass(ones_tab, nidx16, eidx16)

    s_part = _sc_pass(x_node_features, eidx16, nidx16)

    m2 = pl.pallas_call(
        _tc_combine_scale,
        out_shape=jax.ShapeDtypeStruct((NPAD, D), jnp.float32),
    )(s_part[0, :BINS], s_part[1, :BINS],
      ecnt[0, :BINS, 0:1], ecnt[1, :BINS, 0:1], W)

    o_part = _sc_pass(m2, nidx16, eidx16)

    out = pl.pallas_call(
        _tc_finalize,
        out_shape=jax.ShapeDtypeStruct((1, D), jnp.float32),
    )(o_part[0, :BINS], o_part[1, :BINS],
      ncnt[0, :BINS, 0:1], ncnt[1, :BINS, 0:1], b.reshape(1, D))
    return out.reshape(D)


# speedup vs baseline: 3.0182x; 1.3102x over previous
import jax
import jax.numpy as jnp
from jax import lax
from jax.experimental import pallas as pl
from jax.experimental.pallas import tpu as pltpu
from jax.experimental.pallas import tpu_sc as plsc

N_NODES = 10000
N_INC = 320000
D = 128
NC = 2
NS = 16
NW = NC * NS
BINS = 5120
GARB = 128
ROWS = BINS + GARB
NPAD = 2 * BINS
BI = 80
CHUNK = N_INC // NS
NB = CHUNK // BI
STRIPE = ROWS // NS
HCHUNK = N_INC // NW
HNB = HCHUNK // BI
HSTRIPE = NPAD // NS
CW = 16

_mesh = plsc.VectorSubcoreMesh(core_axis_name="c", subcore_axis_name="s")


def _zero_fill_2d(buf, rows, width):
    z16 = jnp.zeros((16,), jnp.float32)

    def body(i, _):
        for k in range(width // 16):
            buf[i, pl.ds(k * 16, 16)] = z16
        return 0

    lax.fori_loop(0, rows, body, 0)


def _sc_aggregate(rows_hbm, widx_hbm, ridx_hbm, acc_out,
                  widx2d, ridx2d, rowbuf, rowbuf2, sem, sem2, acc_sp):
    c = lax.axis_index("c")
    s = lax.axis_index("s")
    lo = c * BINS

    _zero_fill_2d(rowbuf, BI, D)

    def zstripe(r, _):
        pltpu.sync_copy(rowbuf, acc_sp.at[pl.ds(s * STRIPE + r * BI, BI)])
        return 0

    lax.fori_loop(0, STRIPE // BI, zstripe, 0)
    tail = STRIPE - (STRIPE // BI) * BI
    toff = s * STRIPE + (STRIPE // BI) * BI
    if tail:
        pltpu.sync_copy(rowbuf.at[pl.ds(0, tail)],
                        acc_sp.at[pl.ds(toff, tail)])

    pltpu.sync_copy(widx_hbm.at[s], widx2d)
    pltpu.sync_copy(ridx_hbm.at[s], ridx2d)

    def remap(j, _):
        for k in range(BI // 16):
            idx = widx2d[j, pl.ds(k * 16, 16)]
            local = idx - lo
            oob = (local < 0) | (local >= BINS)
            garb = BINS + (idx & (GARB - 1))
            widx2d[j, pl.ds(k * 16, 16)] = jnp.where(oob, garb, local)
        return 0

    lax.fori_loop(0, NB, remap, 0)
    plsc.subcore_barrier()

    def step(g, _):
        j = 2 * g
        dga = pltpu.async_copy(rows_hbm.at[ridx2d.at[j]], rowbuf, sem)
        dgb = pltpu.async_copy(rows_hbm.at[ridx2d.at[j + 1]], rowbuf2, sem2)
        dga.wait()
        dgb.wait()
        dsa = pltpu.async_copy(rowbuf, acc_sp.at[widx2d.at[j]], sem,
                               add=True)
        dsb = pltpu.async_copy(rowbuf2, acc_sp.at[widx2d.at[j + 1]], sem2,
                               add=True)
        dsa.wait()
        dsb.wait()
        return 0

    lax.fori_loop(0, NB // 2, step, 0)
    plsc.subcore_barrier()

    def bounce(r, _):
        pltpu.sync_copy(acc_sp.at[pl.ds(s * STRIPE + r * BI, BI)], rowbuf)
        pltpu.sync_copy(rowbuf, acc_out.at[c, pl.ds(s * STRIPE + r * BI, BI)])
        return 0

    lax.fori_loop(0, STRIPE // BI, bounce, 0)
    if tail:
        pltpu.sync_copy(acc_sp.at[pl.ds(toff, tail)],
                        rowbuf.at[pl.ds(0, tail)])
        pltpu.sync_copy(rowbuf.at[pl.ds(0, tail)],
                        acc_out.at[c, pl.ds(toff, tail)])


_sc_pass = pl.kernel(
    _sc_aggregate,
    out_type=jax.ShapeDtypeStruct((NC, ROWS, D), jnp.float32),
    mesh=_mesh,
    scratch_types=(
        pltpu.VMEM((NB, BI), jnp.int32),
        pltpu.VMEM((NB, BI), jnp.int32),
        pltpu.VMEM((BI, D), jnp.float32),
        pltpu.VMEM((BI, D), jnp.float32),
        pltpu.SemaphoreType.DMA,
        pltpu.SemaphoreType.DMA,
        pltpu.VMEM_SHARED((ROWS, D), jnp.float32),
    ),
)


def _sc_hist(eidx_hbm, nidx_hbm, ecnt_out, ncnt_out,
             eidx2d, nidx2d, ones_v, ecnt_sp, ncnt_sp):
    c = lax.axis_index("c")
    s = lax.axis_index("s")
    w = c * NS + s

    _zero_fill_2d(ones_v, BI, CW)

    def zstripe(r, _):
        pltpu.sync_copy(ones_v, ecnt_sp.at[pl.ds(s * HSTRIPE + r * BI, BI)])
        pltpu.sync_copy(ones_v, ncnt_sp.at[pl.ds(s * HSTRIPE + r * BI, BI)])
        return 0

    lax.fori_loop(0, HSTRIPE // BI, zstripe, 0)

    o16 = jnp.ones((16,), jnp.float32)

    def fill(i, _):
        ones_v[i, pl.ds(0, 16)] = o16
        return 0

    lax.fori_loop(0, BI, fill, 0)

    pltpu.sync_copy(eidx_hbm.at[w], eidx2d)
    pltpu.sync_copy(nidx_hbm.at[w], nidx2d)
    plsc.subcore_barrier()

    def step(j, _):
        pltpu.sync_copy(ones_v, ecnt_sp.at[eidx2d.at[j]], add=True)
        pltpu.sync_copy(ones_v, ncnt_sp.at[nidx2d.at[j]], add=True)
        return 0

    lax.fori_loop(0, HNB, step, 0)
    plsc.subcore_barrier()

    def bounce(r, _):
        off = s * HSTRIPE + r * BI
        pltpu.sync_copy(ecnt_sp.at[pl.ds(off, BI)], ones_v)
        pltpu.sync_copy(ones_v, ecnt_out.at[c, pl.ds(off, BI)])
        pltpu.sync_copy(ncnt_sp.at[pl.ds(off, BI)], ones_v)
        pltpu.sync_copy(ones_v, ncnt_out.at[c, pl.ds(off, BI)])
        return 0

    lax.fori_loop(0, HSTRIPE // BI, bounce, 0)


_sc_hist_pass = pl.kernel(
    _sc_hist,
    out_type=(
        jax.ShapeDtypeStruct((NC, NPAD, CW), jnp.float32),
        jax.ShapeDtypeStruct((NC, NPAD, CW), jnp.float32),
    ),
    mesh=_mesh,
    scratch_types=(
        pltpu.VMEM((HNB, BI), jnp.int32),
        pltpu.VMEM((HNB, BI), jnp.int32),
        pltpu.VMEM((BI, CW), jnp.float32),
        pltpu.VMEM_SHARED((NPAD, CW), jnp.float32),
        pltpu.VMEM_SHARED((NPAD, CW), jnp.float32),
    ),
)


def _tc_combine_scale(s0, s1, c0, c1, w_ref, out_ref):
    ssum = jnp.concatenate([s0[:], s1[:]], axis=0)
    cnt = jnp.concatenate([c0[:], c1[:]], axis=0)
    inv = jnp.where(cnt > 0, 1.0 / jnp.maximum(cnt, 1.0), 0.0)
    m = lax.dot_general(ssum, w_ref[:], (((1,), (1,)), ((), ())),
                        preferred_element_type=jnp.float32)
    out_ref[:] = m * inv


def _tc_finalize(o0, o1, c0, c1, bias_ref, out_ref):
    t = jnp.concatenate([o0[:], o1[:]], axis=0)
    cnt = jnp.concatenate([c0[:], c1[:]], axis=0)
    inv = jnp.where(cnt > 0, 1.0 / jnp.maximum(cnt, 1.0), 0.0)
    t = t * inv + bias_ref[:]
    t = jnp.maximum(t, 0.0)
    rows = lax.broadcasted_iota(jnp.int32, (NPAD, D), 0)
    t = jnp.where(rows < N_NODES, t, 0.0)
    out_ref[:] = jnp.sum(t, axis=0, keepdims=True) * (1.0 / N_NODES)


def kernel(x_node_features, hyperedge_index, W, b):
    node_idx = hyperedge_index[0].astype(jnp.int32)
    edge_idx = hyperedge_index[1].astype(jnp.int32)
    nidx16 = node_idx.reshape(NS, NB, BI)
    eidx16 = edge_idx.reshape(NS, NB, BI)
    nidx32 = node_idx.reshape(NW, HNB, BI)
    eidx32 = edge_idx.reshape(NW, HNB, BI)

    ones_tab = jnp.ones((N_NODES, D), jnp.float32)
    ecnt = _sc_pass(ones_tab, eidx16, nidx16)
    ncnt = _sc_pass(ones_tab, nidx16, eidx16)

    s_part = _sc_pass(x_node_features, eidx16, nidx16)

    m2 = pl.pallas_call(
        _tc_combine_scale,
        out_shape=jax.ShapeDtypeStruct((NPAD, D), jnp.float32),
    )(s_part[0, :BINS], s_part[1, :BINS],
      ecnt[0, :BINS, 0:1], ecnt[1, :BINS, 0:1], W)

    o_part = _sc_pass(m2, nidx16, eidx16)

    out = pl.pallas_call(
        _tc_finalize,
        out_shape=jax.ShapeDtypeStruct((1, D), jnp.float32),
    )(o_part[0, :BINS], o_part[1, :BINS],
      ncnt[0, :BINS, 0:1], ncnt[1, :BINS, 0:1], b.reshape(1, D))
    return out.reshape(D)
